# baseline (speedup 1.0000x reference)
import functools

import jax
import jax.numpy as jnp
import numpy as np
from jax import lax
from jax.experimental import pallas as pl
from jax.experimental.pallas import tpu as pltpu
from jax.experimental.pallas import tpu_sc as plsc

D = 64
LW = 50
NC = 2
NS = 16
NW = NC * NS
LANES = 16

_PERM = np.array(
    [32 * k + 2 * i + o for k in range(D // 32) for o in (0, 1)
     for i in range(16)], dtype=np.int32)


def _bow_sc(idx, emb_bf, B):
    per_w = B // NW
    chunk = 16
    n_chunks = per_w // chunk
    rpc = chunk * LW

    mesh = plsc.VectorSubcoreMesh(core_axis_name="c", subcore_axis_name="s")

    NB = 3
    HI = jnp.uint32(0xFFFF0000)

    @functools.partial(
        pl.kernel,
        mesh=mesh,
        compiler_params=pltpu.CompilerParams(
            use_tc_tiling_on_sc=False, needs_layout_passes=False),
        out_type=jax.ShapeDtypeStruct((B, D), jnp.float32),
        scratch_types=[
            pltpu.VMEM((per_w * LW,), jnp.int32),
            pltpu.VMEM((rpc, D), jnp.bfloat16),
            pltpu.VMEM((rpc, D), jnp.bfloat16),
            pltpu.VMEM((rpc, D), jnp.bfloat16),
            pltpu.VMEM((chunk, D), jnp.float32),
            pltpu.VMEM((chunk, D), jnp.float32),
            pltpu.SemaphoreType.DMA,
            pltpu.SemaphoreType.DMA,
            pltpu.SemaphoreType.DMA,
            pltpu.SemaphoreType.DMA,
            pltpu.SemaphoreType.DMA,
        ],
    )
    def bow_kernel(idx_hbm, table_hbm, out_hbm,
                   idx_v, rows0, rows1, rows2, outv0, outv1,
                   gsem0, gsem1, gsem2, osem0, osem1):
        wid = lax.axis_index("s") * NC + lax.axis_index("c")
        base = wid * per_w
        row_bufs = (rows0, rows1, rows2)
        out_bufs = (outv0, outv1)
        gsems = (gsem0, gsem1, gsem2)
        osems = (osem0, osem1)

        pltpu.sync_copy(idx_hbm.at[pl.ds(base * LW, per_w * LW)], idx_v)

        def fetch(c, p):
            pltpu.async_copy(
                table_hbm.at[idx_v.at[pl.ds(c * rpc, rpc)]],
                row_bufs[p], gsems[p])

        def wait_gather(c, p):
            pltpu.make_async_copy(
                table_hbm.at[idx_v.at[pl.ds(c * rpc, rpc)]],
                row_bufs[p], gsems[p]).wait()

        def store_out(c, p):
            pltpu.async_copy(
                out_bufs[p], out_hbm.at[pl.ds(base + c * chunk, chunk)],
                osems[p])

        def wait_store(c, p):
            pltpu.make_async_copy(
                out_bufs[p], out_hbm.at[pl.ds(base + c * chunk, chunk)],
                osems[p]).wait()

        def compute(c, gp, op):
            rows_v = row_bufs[gp]
            out_v = out_bufs[op]

            def row_body(i, carry):
                rbase = i * LW
                accs = [None] * 4

                def addrow(r, accs):
                    new = []
                    for k in range(D // 32):
                        raw = rows_v[r, pl.ds(k * 32, 32)]
                        v = plsc.bitcast(raw, jnp.uint32)
                        even = plsc.bitcast(
                            lax.shift_left(v, jnp.uint32(16)), jnp.float32)
                        odd = plsc.bitcast(
                            lax.bitwise_and(v, HI), jnp.float32)
                        e0 = accs[2 * k]
                        o0 = accs[2 * k + 1]
                        new.append(even if e0 is None else e0 + even)
                        new.append(odd if o0 is None else o0 + odd)
                    return new

                accs = addrow(rbase, accs)
                for j in range(1, LW):
                    accs = addrow(rbase + j, accs)
                for t in range(4):
                    out_v[i, pl.ds(t * LANES, LANES)] = accs[t]
                return carry

            lax.fori_loop(0, chunk, row_body, 0)

        for c0 in range(NB):
            fetch(c0, c0)

        def step(c, gp, op, do_waitstore=True):
            wait_gather(c, gp)
            if do_waitstore:
                wait_store(c - 2, op)
            compute(c, gp, op)
            if isinstance(c, int):
                if c + NB < n_chunks:
                    fetch(c + NB, gp)
            else:
                @pl.when(c + NB < n_chunks)
                def _():
                    fetch(c + NB, gp)
            store_out(c, op)

        GRP = NB * 2
        n_main = n_chunks // GRP

        for q in range(GRP):
            step(q, q % NB, q % 2, q >= 2)

        def group_body(h, carry):
            for q in range(GRP):
                c = h * GRP + q
                step(c, q % NB, q % 2)
            return carry

        lax.fori_loop(1, n_main, group_body, 0)

        for c in range(n_main * GRP, n_chunks):
            step(c, c % NB, c % 2)
        for p in range(2):
            wait_store(n_chunks - 2 + p, p)

    return bow_kernel(idx, emb_bf)


def _dense_softmax(bow, img, W, b, B, OUT):
    blk = 512
    grid = B // blk
    IMG = img.shape[1]

    def body(bow_ref, img_ref, w_ref, b_ref, out_ref):
        w1 = w_ref[pl.ds(0, D), :]
        w2 = w_ref[pl.ds(D, IMG), :]
        dn = (((0,), (1,)), ((), ()))
        logits = lax.dot_general(w1, bow_ref[...], dn,
                                 preferred_element_type=jnp.float32)
        logits = logits + lax.dot_general(w2, img_ref[...], dn,
                                          preferred_element_type=jnp.float32)
        logits = logits + b_ref[...]
        m = jnp.max(logits, axis=0, keepdims=True)
        e = jnp.exp(logits - m)
        s = jnp.sum(e, axis=0, keepdims=True)
        out_ref[...] = e / s

    out_t = pl.pallas_call(
        body,
        grid=(grid,),
        in_specs=[
            pl.BlockSpec((blk, D), lambda i: (i, 0)),
            pl.BlockSpec((blk, IMG), lambda i: (i, 0)),
            pl.BlockSpec((D + IMG, OUT), lambda i: (0, 0)),
            pl.BlockSpec((OUT, 1), lambda i: (0, 0)),
        ],
        out_specs=pl.BlockSpec((OUT, blk), lambda i: (0, i)),
        out_shape=jax.ShapeDtypeStruct((OUT, B), jnp.float32),
    )(bow, img, W, b)
    return out_t.T


@jax.jit
def kernel(word_features, image_features, emb_table, W, b):
    B, L = word_features.shape
    assert L == LW
    OUT = W.shape[1]
    idx = word_features.astype(jnp.int32).reshape(-1)
    emb_bf = (emb_table.reshape(-1)
              .astype(jnp.bfloat16)
              .reshape(emb_table.shape))
    bow = _bow_sc(idx, emb_bf, B)
    Wp = jnp.concatenate([W[:D][jnp.asarray(_PERM)], W[D:]], axis=0)
    return _dense_softmax(bow, image_features, Wp, b.reshape(-1, 1), B, OUT)


# speedup vs baseline: 1.1202x; 1.1202x over previous
import functools

import jax
import jax.numpy as jnp
from jax import lax
from jax.experimental import pallas as pl
from jax.experimental.pallas import tpu as pltpu
from jax.experimental.pallas import tpu_sc as plsc

D = 64
LW = 50
NC = 2
NS = 16
NW = NC * NS
LANES = 16


def _bow_sc(idx, emb_table, B):
    per_w = B // NW
    chunk = 8
    n_chunks = per_w // chunk
    rpc = chunk * LW
    KS = D // LANES

    mesh = plsc.VectorSubcoreMesh(core_axis_name="c", subcore_axis_name="s")

    NB = 3

    @functools.partial(
        pl.kernel,
        mesh=mesh,
        compiler_params=pltpu.CompilerParams(use_tc_tiling_on_sc=False),
        out_type=jax.ShapeDtypeStruct((B, D), jnp.float32),
        scratch_types=[
            pltpu.VMEM((per_w * LW,), jnp.int32),
            pltpu.VMEM((rpc, D), jnp.float32),
            pltpu.VMEM((rpc, D), jnp.float32),
            pltpu.VMEM((rpc, D), jnp.float32),
            pltpu.VMEM((chunk, D), jnp.float32),
            pltpu.VMEM((chunk, D), jnp.float32),
            pltpu.SemaphoreType.DMA,
            pltpu.SemaphoreType.DMA,
            pltpu.SemaphoreType.DMA,
            pltpu.SemaphoreType.DMA,
            pltpu.SemaphoreType.DMA,
        ],
    )
    def bow_kernel(idx_hbm, table_hbm, out_hbm,
                   idx_v, rows0, rows1, rows2, outv0, outv1,
                   gsem0, gsem1, gsem2, osem0, osem1):
        wid = lax.axis_index("s") * NC + lax.axis_index("c")
        base = wid * per_w
        row_bufs = (rows0, rows1, rows2)
        out_bufs = (outv0, outv1)
        gsems = (gsem0, gsem1, gsem2)
        osems = (osem0, osem1)

        pltpu.sync_copy(idx_hbm.at[pl.ds(base * LW, per_w * LW)], idx_v)

        def fetch(c, p):
            pltpu.async_copy(
                table_hbm.at[idx_v.at[pl.ds(c * rpc, rpc)]],
                row_bufs[p], gsems[p])

        def wait_gather(c, p):
            pltpu.make_async_copy(
                table_hbm.at[idx_v.at[pl.ds(c * rpc, rpc)]],
                row_bufs[p], gsems[p]).wait()

        def store_out(c, p):
            pltpu.async_copy(
                out_bufs[p], out_hbm.at[pl.ds(base + c * chunk, chunk)],
                osems[p])

        def wait_store(c, p):
            pltpu.make_async_copy(
                out_bufs[p], out_hbm.at[pl.ds(base + c * chunk, chunk)],
                osems[p]).wait()

        def compute(c, gp, op):
            rows_v = row_bufs[gp]
            out_v = out_bufs[op]

            def row_body(i, carry):
                rbase = i * LW
                accs = [rows_v[rbase, pl.ds(k * LANES, LANES)]
                        for k in range(KS)]
                for j in range(1, LW):
                    for k in range(KS):
                        accs[k] = accs[k] + rows_v[
                            rbase + j, pl.ds(k * LANES, LANES)]
                for k in range(KS):
                    out_v[i, pl.ds(k * LANES, LANES)] = accs[k]
                return carry

            lax.fori_loop(0, chunk, row_body, 0)

        for c0 in range(NB):
            fetch(c0, c0)

        def step(c, gp, op, do_waitstore=True):
            wait_gather(c, gp)
            if do_waitstore:
                wait_store(c - 2, op)
            compute(c, gp, op)
            if isinstance(c, int):
                if c + NB < n_chunks:
                    fetch(c + NB, gp)
            else:
                @pl.when(c + NB < n_chunks)
                def _():
                    fetch(c + NB, gp)
            store_out(c, op)

        GRP = NB * 2
        n_main = n_chunks // GRP

        for q in range(GRP):
            step(q, q % NB, q % 2, q >= 2)

        def group_body(h, carry):
            for q in range(GRP):
                c = h * GRP + q
                step(c, q % NB, q % 2)
            return carry

        lax.fori_loop(1, n_main, group_body, 0)

        for c in range(n_main * GRP, n_chunks):
            step(c, c % NB, c % 2)
        for p in range(2):
            wait_store(n_chunks - 2 + p, p)

    return bow_kernel(idx, emb_table)


def _dense_softmax(bow, img, W, b, B, OUT):
    blk = 512
    grid = B // blk
    IMG = img.shape[1]

    def body(bow_ref, img_ref, w_ref, b_ref, out_ref):
        w1 = w_ref[pl.ds(0, D), :]
        w2 = w_ref[pl.ds(D, IMG), :]
        dn = (((0,), (1,)), ((), ()))
        logits = lax.dot_general(w1, bow_ref[...], dn,
                                 preferred_element_type=jnp.float32)
        logits = logits + lax.dot_general(w2, img_ref[...], dn,
                                          preferred_element_type=jnp.float32)
        logits = logits + b_ref[...]
        m = jnp.max(logits, axis=0, keepdims=True)
        e = jnp.exp(logits - m)
        s = jnp.sum(e, axis=0, keepdims=True)
        out_ref[...] = e / s

    out_t = pl.pallas_call(
        body,
        grid=(grid,),
        in_specs=[
            pl.BlockSpec((blk, D), lambda i: (i, 0)),
            pl.BlockSpec((blk, IMG), lambda i: (i, 0)),
            pl.BlockSpec((D + IMG, OUT), lambda i: (0, 0)),
            pl.BlockSpec((OUT, 1), lambda i: (0, 0)),
        ],
        out_specs=pl.BlockSpec((OUT, blk), lambda i: (0, i)),
        out_shape=jax.ShapeDtypeStruct((OUT, B), jnp.float32),
    )(bow, img, W, b)
    return out_t.T


@jax.jit
def kernel(word_features, image_features, emb_table, W, b):
    B, L = word_features.shape
    assert L == LW
    OUT = W.shape[1]
    idx = word_features.astype(jnp.int32).reshape(-1)
    bow = _bow_sc(idx, emb_table, B)
    return _dense_softmax(bow, image_features, W, b.reshape(-1, 1), B, OUT)


# speedup vs baseline: 1.1715x; 1.0458x over previous
import functools

import jax
import jax.numpy as jnp
from jax import lax
from jax.experimental import pallas as pl
from jax.experimental.pallas import tpu as pltpu
from jax.experimental.pallas import tpu_sc as plsc

D = 64
LW = 50
NC = 2
NS = 16
NW = NC * NS
LANES = 16


def _bow_sc(idx, emb_table, B):
    per_w = B // NW
    chunk = 8
    n_chunks = per_w // chunk
    rpc = chunk * LW
    KS = D // LANES

    mesh = plsc.VectorSubcoreMesh(core_axis_name="c", subcore_axis_name="s")

    NB = 3

    @functools.partial(
        pl.kernel,
        mesh=mesh,
        compiler_params=pltpu.CompilerParams(use_tc_tiling_on_sc=False),
        out_type=jax.ShapeDtypeStruct((B, D), jnp.float32),
        scratch_types=[
            pltpu.VMEM((per_w * LW,), jnp.int32),
            pltpu.VMEM((rpc, D), jnp.float32),
            pltpu.VMEM((rpc, D), jnp.float32),
            pltpu.VMEM((rpc, D), jnp.float32),
            pltpu.VMEM((chunk, D), jnp.float32),
            pltpu.VMEM((chunk, D), jnp.float32),
            pltpu.SemaphoreType.DMA,
            pltpu.SemaphoreType.DMA,
            pltpu.SemaphoreType.DMA,
            pltpu.SemaphoreType.DMA,
            pltpu.SemaphoreType.DMA,
        ],
    )
    def bow_kernel(idx_hbm, table_hbm, out_hbm,
                   idx_v, rows0, rows1, rows2, outv0, outv1,
                   gsem0, gsem1, gsem2, osem0, osem1):
        wid = lax.axis_index("s") * NC + lax.axis_index("c")
        base = wid * per_w
        row_bufs = (rows0, rows1, rows2)
        out_bufs = (outv0, outv1)
        gsems = (gsem0, gsem1, gsem2)
        osems = (osem0, osem1)

        pltpu.sync_copy(idx_hbm.at[pl.ds(base * LW, per_w * LW)], idx_v)

        def fetch(c, p):
            pltpu.async_copy(
                table_hbm.at[idx_v.at[pl.ds(c * rpc, rpc)]],
                row_bufs[p], gsems[p])

        def wait_gather(c, p):
            pltpu.make_async_copy(
                table_hbm.at[idx_v.at[pl.ds(c * rpc, rpc)]],
                row_bufs[p], gsems[p]).wait()

        def store_out(c, p):
            pltpu.async_copy(
                out_bufs[p], out_hbm.at[pl.ds(base + c * chunk, chunk)],
                osems[p])

        def wait_store(c, p):
            pltpu.make_async_copy(
                out_bufs[p], out_hbm.at[pl.ds(base + c * chunk, chunk)],
                osems[p]).wait()

        def compute(c, gp, op):
            rows_v = row_bufs[gp]
            out_v = out_bufs[op]

            def row_body(i, carry):
                rbase = i * LW
                accs = [rows_v[rbase, pl.ds(k * LANES, LANES)]
                        for k in range(KS)]
                for j in range(1, LW):
                    for k in range(KS):
                        accs[k] = accs[k] + rows_v[
                            rbase + j, pl.ds(k * LANES, LANES)]
                for k in range(KS):
                    out_v[i, pl.ds(k * LANES, LANES)] = accs[k]
                return carry

            lax.fori_loop(0, chunk, row_body, 0)

        for c0 in range(NB):
            fetch(c0, c0)

        def step(c, gp, op, do_waitstore=True):
            wait_gather(c, gp)
            if do_waitstore:
                wait_store(c - 2, op)
            compute(c, gp, op)
            if isinstance(c, int):
                if c + NB < n_chunks:
                    fetch(c + NB, gp)
            else:
                @pl.when(c + NB < n_chunks)
                def _():
                    fetch(c + NB, gp)
            store_out(c, op)

        GRP = NB * 2
        n_main = n_chunks // GRP

        for q in range(GRP):
            step(q, q % NB, q % 2, q >= 2)

        def group_body(h, carry):
            for q in range(GRP):
                c = h * GRP + q
                step(c, q % NB, q % 2)
            return carry

        lax.fori_loop(1, n_main, group_body, 0)

        for c in range(n_main * GRP, n_chunks):
            step(c, c % NB, c % 2)
        for p in range(2):
            wait_store(n_chunks - 2 + p, p)

    return bow_kernel(idx, emb_table)


def _dense_softmax(bow, img, W, b, B, OUT):
    blk = 1024
    grid = B // blk
    IMG = img.shape[1]

    def body(bow_ref, img_ref, w_ref, b_ref, out_ref):
        w1 = w_ref[pl.ds(0, D), :]
        w2 = w_ref[pl.ds(D, IMG), :]
        dn = (((0,), (1,)), ((), ()))
        logits = lax.dot_general(w1, bow_ref[...], dn,
                                 preferred_element_type=jnp.float32)
        logits = logits + lax.dot_general(w2, img_ref[...], dn,
                                          preferred_element_type=jnp.float32)
        logits = logits + b_ref[...]
        m = jnp.max(logits, axis=0, keepdims=True)
        e = jnp.exp(logits - m)
        s = jnp.sum(e, axis=0, keepdims=True)
        out_ref[...] = e / s

    out_t = pl.pallas_call(
        body,
        grid=(grid,),
        in_specs=[
            pl.BlockSpec((blk, D), lambda i: (i, 0)),
            pl.BlockSpec((blk, IMG), lambda i: (i, 0)),
            pl.BlockSpec((D + IMG, OUT), lambda i: (0, 0)),
            pl.BlockSpec((OUT, 1), lambda i: (0, 0)),
        ],
        out_specs=pl.BlockSpec((OUT, blk), lambda i: (0, i)),
        out_shape=jax.ShapeDtypeStruct((OUT, B), jnp.float32),
    )(bow, img, W, b)
    return out_t.T


@jax.jit
def kernel(word_features, image_features, emb_table, W, b):
    B, L = word_features.shape
    assert L == LW
    OUT = W.shape[1]
    idx = word_features.astype(jnp.int32).reshape(-1)
    bow = _bow_sc(idx, emb_table, B)
    return _dense_softmax(bow, image_features, W, b.reshape(-1, 1), B, OUT)
